# baseline (speedup 1.0000x reference)
import functools

import jax
import jax.numpy as jnp
from jax import lax
from jax.experimental import pallas as pl
from jax.experimental.pallas import tpu as pltpu
from jax.experimental.pallas import tpu_sc as plsc

_NC = 2
_NS = 16
_NW = _NC * _NS
_LANES = 16


def _film_body(od, nf_ref, cond_ref, wt_ref, pt_ref, Wc_ref, bc_ref, Wf_ref,
               bf_ref, tbl_ref, ew_ref):
    bpg, N, D = nf_ref.shape
    K = wt_ref.shape[1]
    nf = nf_ref[...].reshape(bpg * N, D)
    x = lax.dot_general(nf, Wf_ref[...], (((1,), (1,)), ((), ())),
                        preferred_element_type=jnp.float32)
    x = x + bf_ref[...]
    mu = jnp.mean(x, axis=1, keepdims=True)
    xc = x - mu
    var = jnp.mean(xc * xc, axis=1, keepdims=True)
    xn = xc / jnp.sqrt(var + 1e-5)
    cond = cond_ref[...].reshape(bpg, cond_ref.shape[2])
    gb = lax.dot_general(cond, Wc_ref[...], (((1,), (1,)), ((), ())),
                         preferred_element_type=jnp.float32)
    gb = gb + bc_ref[...]
    gamma = (gb[:, :od] + 1.0)[:, None, :]
    beta = gb[:, od:][:, None, :]
    gfull = jnp.broadcast_to(gamma, (bpg, N, od)).reshape(bpg * N, od)
    bfull = jnp.broadcast_to(beta, (bpg, N, od)).reshape(bpg * N, od)
    tbl_ref[...] = jnp.maximum(gfull * xn + bfull, 0.0)
    rows_per_b = (N * K) // 128
    for bb in range(bpg):
        ew = wt_ref[bb] * pt_ref[bb]
        ewT = ew.T
        ew3 = ewT.reshape(N // 4, 4, K)
        for j in range(4):
            ew_ref[pl.ds(bb * rows_per_b, rows_per_b),
                   pl.ds(j * K, K)] = ew3[:, j, :]


def _film_call(node_feats, cond_feats, w_t, p_t, W_cond, b_cond, W_film,
               b_film):
    B, N, D = node_feats.shape
    K = w_t.shape[1]
    OD = W_film.shape[0]
    CD = W_cond.shape[1]
    G = 1
    BPG = B // G
    RPG = (BPG * N * K) // 128
    return pl.pallas_call(
        functools.partial(_film_body, OD),
        grid=(G,),
        in_specs=[
            pl.BlockSpec((BPG, N, D), lambda b: (b, 0, 0)),
            pl.BlockSpec((BPG, 1, CD), lambda b: (b, 0, 0)),
            pl.BlockSpec((BPG, K, N), lambda b: (b, 0, 0)),
            pl.BlockSpec((BPG, K, N), lambda b: (b, 0, 0)),
            pl.BlockSpec((2 * OD, CD), lambda b: (0, 0)),
            pl.BlockSpec((1, 2 * OD), lambda b: (0, 0)),
            pl.BlockSpec((OD, D), lambda b: (0, 0)),
            pl.BlockSpec((1, OD), lambda b: (0, 0)),
        ],
        out_specs=[
            pl.BlockSpec((BPG * N, OD), lambda b: (b, 0)),
            pl.BlockSpec((RPG, 128), lambda b: (b, 0)),
        ],
        out_shape=[
            jax.ShapeDtypeStruct((B * N, OD), jnp.float32),
            jax.ShapeDtypeStruct(((B * N * K) // 128, 128), jnp.float32),
        ],
    )(node_feats, cond_feats, w_t, p_t, W_cond, b_cond.reshape(1, 2 * OD),
      W_film, b_film.reshape(1, OD))


def _make_sc_gather(n_nodes, OD, K, E):
    CE = 128
    CN = CE // K
    NCH = OD // _LANES
    n_chunks = E // CE
    assert n_chunks * CE == E
    SMALL = (n_chunks // _NW) // 8 * 8
    blocks8 = n_chunks // 8
    NBIG = blocks8 - _NW * (SMALL // 8)
    BIG = SMALL + 8
    LEFT = n_chunks - 8 * blocks8
    NM_LAST = SMALL + LEFT
    assert 0 <= NBIG < _NW
    assert SMALL > 0 and LEFT % 2 == 0
    HALF = (BIG // 2 + 7) // 8 * 8
    assert SMALL > HALF and BIG - HALF <= HALF
    CAP = BIG
    mesh = plsc.VectorSubcoreMesh(core_axis_name="c", subcore_axis_name="s")

    @functools.partial(
        pl.kernel,
        out_type=jax.ShapeDtypeStruct((n_nodes, OD), jnp.float32),
        mesh=mesh,
        scratch_types=[
            pltpu.VMEM((CAP * CE,), jnp.int32),
            pltpu.VMEM((CAP, CE), jnp.float32),
            pltpu.VMEM((5, CE, OD), jnp.float32),
            pltpu.VMEM((HALF * CN, OD), jnp.float32),
            pltpu.SemaphoreType.DMA((5,)),
        ],
    )
    def sc_gather(tbl_hbm, idx_hbm, ew_hbm, out_hbm, idx_v, ew_v, rows_v,
                  out_v, sem):
        wid = lax.axis_index("s") * _NC + lax.axis_index("c")
        is_big = wid < NBIG
        is_last = wid == _NW - 1
        start = jnp.where(is_big, wid * BIG,
                          NBIG * BIG + (wid - NBIG) * SMALL)
        n_mine = jnp.where(is_big, BIG,
                           jnp.where(is_last, NM_LAST, SMALL))
        ebase = start * CE

        pltpu.sync_copy(idx_hbm.at[pl.ds(ebase, SMALL * CE)],
                        idx_v.at[pl.ds(0, SMALL * CE)])
        pltpu.sync_copy(ew_hbm.at[pl.ds(start, SMALL)],
                        ew_v.at[pl.ds(0, SMALL)])

        @pl.when(is_big)
        def _():
            pltpu.sync_copy(idx_hbm.at[pl.ds(ebase + SMALL * CE, 8 * CE)],
                            idx_v.at[pl.ds(SMALL * CE, 8 * CE)])
            pltpu.sync_copy(ew_hbm.at[pl.ds(start + SMALL, 8)],
                            ew_v.at[pl.ds(SMALL, 8)])

        if LEFT:
            @pl.when(is_last)
            def _():
                pltpu.sync_copy(
                    idx_hbm.at[pl.ds(ebase + SMALL * CE, LEFT * CE)],
                    idx_v.at[pl.ds(SMALL * CE, LEFT * CE)])
                pltpu.sync_copy(ew_hbm.at[pl.ds(start + SMALL, LEFT)],
                                ew_v.at[pl.ds(SMALL, LEFT)])

        lane_splat = [jnp.full((_LANES,), j, jnp.int32) for j in range(_LANES)]

        H = CE // 2

        def issue(ci, slot):
            pltpu.async_copy(tbl_hbm.at[idx_v.at[pl.ds(ci * CE, H)]],
                             rows_v.at[slot, pl.ds(0, H)], sem.at[slot])
            pltpu.async_copy(tbl_hbm.at[idx_v.at[pl.ds(ci * CE + H, H)]],
                             rows_v.at[slot, pl.ds(H, H)], sem.at[slot])

        def wait(slot):
            pltpu.make_async_copy(tbl_hbm.at[idx_v.at[pl.ds(0, H)]],
                                  rows_v.at[slot, pl.ds(0, H)],
                                  sem.at[slot]).wait()
            pltpu.make_async_copy(tbl_hbm.at[idx_v.at[pl.ds(0, H)]],
                                  rows_v.at[slot, pl.ds(H, H)],
                                  sem.at[slot]).wait()

        def compute(ci, slot):
            lrow = (ci - jnp.where(ci >= HALF, HALF, 0)) * CN
            for q in range(CN):
                acc = [jnp.zeros((_LANES,), jnp.float32) for _ in range(NCH)]
                for g in range(K // _LANES):
                    off = q * K + g * _LANES
                    ew = ew_v[ci, pl.ds(off, _LANES)]
                    for jj in range(_LANES):
                        e = q * K + g * _LANES + jj
                        wb = ew.at[lane_splat[jj]].get(
                            mode='promise_in_bounds')
                        for c in range(NCH):
                            r = rows_v[slot, e, pl.ds(c * _LANES, _LANES)]
                            acc[c] = acc[c] + wb * r
                row = lrow + q
                for c in range(NCH):
                    out_v[row, pl.ds(c * _LANES, _LANES)] = jnp.maximum(
                        acc[c], 0.0)

        issue(0, 0)
        issue(1, 1)
        issue(2, 2)
        issue(3, 3)

        def gbody(i, carry):
            @pl.when(i == HALF)
            def _():
                pltpu.sync_copy(out_v.at[pl.ds(0, HALF * CN)],
                                out_hbm.at[pl.ds(start * CN, HALF * CN)])

            slot = lax.rem(i, 5)
            wait(slot)
            compute(i, slot)
            nci = i + 4

            @pl.when(nci < n_mine)
            def _():
                issue(nci, lax.rem(nci, 5))
            return carry

        lax.fori_loop(0, n_mine, gbody, 0)

        tail_base = (start + HALF) * CN

        @pl.when(is_big)
        def _():
            pltpu.sync_copy(out_v.at[pl.ds(0, (BIG - HALF) * CN)],
                            out_hbm.at[pl.ds(tail_base, (BIG - HALF) * CN)])

        @pl.when(jnp.logical_and(jnp.logical_not(is_big),
                                 jnp.logical_not(is_last)))
        def _():
            pltpu.sync_copy(out_v.at[pl.ds(0, (SMALL - HALF) * CN)],
                            out_hbm.at[pl.ds(tail_base, (SMALL - HALF) * CN)])

        @pl.when(is_last)
        def _():
            pltpu.sync_copy(
                out_v.at[pl.ds(0, (NM_LAST - HALF) * CN)],
                out_hbm.at[pl.ds(tail_base, (NM_LAST - HALF) * CN)])

    return sc_gather


def kernel(node_feats, cond_feats, weights, params, coords_j, W_cond, b_cond,
           W_film, b_film):
    B, N, D = node_feats.shape
    K = weights.shape[2]
    OD = W_film.shape[0]
    E = B * N * K

    w_t = weights.transpose(0, 2, 3, 1).reshape(B, K, N)
    p_t = params.transpose(0, 2, 3, 1).reshape(B, K, N)
    tbl, ew = _film_call(node_feats, cond_feats, w_t, p_t, W_cond, b_cond,
                         W_film, b_film)
    sc = _make_sc_gather(B * N, OD, K, E)
    idx = (coords_j if coords_j.dtype == jnp.int32
           else coords_j.astype(jnp.int32))
    out = sc(tbl, idx, ew)
    return out.reshape(B, N, OD)


# speedup vs baseline: 1.0122x; 1.0122x over previous
import functools

import jax
import jax.numpy as jnp
from jax import lax
from jax.experimental import pallas as pl
from jax.experimental.pallas import tpu as pltpu
from jax.experimental.pallas import tpu_sc as plsc

_NC = 2
_NS = 16
_NW = _NC * _NS
_LANES = 16


def _film_body(od, nf_ref, cond_ref, wt_ref, pt_ref, Wc_ref, bc_ref, Wf_ref,
               bf_ref, tbl_ref, ew_ref):
    bpg, N, D = nf_ref.shape
    K = wt_ref.shape[1]
    nf = nf_ref[...].reshape(bpg * N, D)
    x = lax.dot_general(nf, Wf_ref[...], (((1,), (1,)), ((), ())),
                        preferred_element_type=jnp.float32)
    x = x + bf_ref[...]
    mu = jnp.mean(x, axis=1, keepdims=True)
    xc = x - mu
    var = jnp.mean(xc * xc, axis=1, keepdims=True)
    xn = xc * lax.rsqrt(var + 1e-5)
    cond = cond_ref[...].reshape(bpg, cond_ref.shape[2])
    gb = lax.dot_general(cond, Wc_ref[...], (((1,), (1,)), ((), ())),
                         preferred_element_type=jnp.float32)
    gb = gb + bc_ref[...]
    gamma = (gb[:, :od] + 1.0)[:, None, :]
    beta = gb[:, od:][:, None, :]
    gfull = jnp.broadcast_to(gamma, (bpg, N, od)).reshape(bpg * N, od)
    bfull = jnp.broadcast_to(beta, (bpg, N, od)).reshape(bpg * N, od)
    tbl_ref[...] = jnp.maximum(gfull * xn + bfull, 0.0)
    rows_per_b = (N * K) // 128
    eye_k = jnp.eye(K, dtype=jnp.float32)
    for bb in range(bpg):
        ew = wt_ref[bb] * pt_ref[bb]
        ewT = lax.dot_general(ew, eye_k, (((0,), (0,)), ((), ())),
                              preferred_element_type=jnp.float32)
        ew3 = ewT.reshape(N // 4, 4, K)
        for j in range(4):
            ew_ref[pl.ds(bb * rows_per_b, rows_per_b),
                   pl.ds(j * K, K)] = ew3[:, j, :]


def _film_call(node_feats, cond_feats, w_t, p_t, W_cond, b_cond, W_film,
               b_film):
    B, N, D = node_feats.shape
    K = w_t.shape[1]
    OD = W_film.shape[0]
    CD = W_cond.shape[1]
    G = 1
    BPG = B // G
    RPG = (BPG * N * K) // 128
    return pl.pallas_call(
        functools.partial(_film_body, OD),
        grid=(G,),
        in_specs=[
            pl.BlockSpec((BPG, N, D), lambda b: (b, 0, 0)),
            pl.BlockSpec((BPG, 1, CD), lambda b: (b, 0, 0)),
            pl.BlockSpec((BPG, K, N), lambda b: (b, 0, 0)),
            pl.BlockSpec((BPG, K, N), lambda b: (b, 0, 0)),
            pl.BlockSpec((2 * OD, CD), lambda b: (0, 0)),
            pl.BlockSpec((1, 2 * OD), lambda b: (0, 0)),
            pl.BlockSpec((OD, D), lambda b: (0, 0)),
            pl.BlockSpec((1, OD), lambda b: (0, 0)),
        ],
        out_specs=[
            pl.BlockSpec((BPG * N, OD), lambda b: (b, 0)),
            pl.BlockSpec((RPG, 128), lambda b: (b, 0)),
        ],
        out_shape=[
            jax.ShapeDtypeStruct((B * N, OD), jnp.float32),
            jax.ShapeDtypeStruct(((B * N * K) // 128, 128), jnp.float32),
        ],
    )(node_feats, cond_feats, w_t, p_t, W_cond, b_cond.reshape(1, 2 * OD),
      W_film, b_film.reshape(1, OD))


def _make_sc_gather(n_nodes, OD, K, E):
    CE = 128
    CN = CE // K
    NCH = OD // _LANES
    n_chunks = E // CE
    assert n_chunks * CE == E
    SMALL = (n_chunks // _NW) // 8 * 8
    blocks8 = n_chunks // 8
    NBIG = blocks8 - _NW * (SMALL // 8)
    BIG = SMALL + 8
    LEFT = n_chunks - 8 * blocks8
    NM_LAST = SMALL + LEFT
    assert 0 <= NBIG < _NW
    assert SMALL > 0 and LEFT % 2 == 0
    HALF = (BIG // 2 + 7) // 8 * 8
    assert SMALL > HALF and BIG - HALF <= HALF
    CAP = BIG
    mesh = plsc.VectorSubcoreMesh(core_axis_name="c", subcore_axis_name="s")

    @functools.partial(
        pl.kernel,
        out_type=jax.ShapeDtypeStruct((n_nodes, OD), jnp.float32),
        mesh=mesh,
        scratch_types=[
            pltpu.VMEM((CAP * CE,), jnp.int32),
            pltpu.VMEM((CAP, CE), jnp.float32),
            pltpu.VMEM((5, CE, OD), jnp.float32),
            pltpu.VMEM((HALF * CN, OD), jnp.float32),
            pltpu.SemaphoreType.DMA((5,)),
        ],
    )
    def sc_gather(tbl_hbm, idx_hbm, ew_hbm, out_hbm, idx_v, ew_v, rows_v,
                  out_v, sem):
        wid = lax.axis_index("s") * _NC + lax.axis_index("c")
        is_big = wid < NBIG
        is_last = wid == _NW - 1
        start = jnp.where(is_big, wid * BIG,
                          NBIG * BIG + (wid - NBIG) * SMALL)
        n_mine = jnp.where(is_big, BIG,
                           jnp.where(is_last, NM_LAST, SMALL))
        ebase = start * CE

        pltpu.sync_copy(idx_hbm.at[pl.ds(ebase, SMALL * CE)],
                        idx_v.at[pl.ds(0, SMALL * CE)])
        pltpu.sync_copy(ew_hbm.at[pl.ds(start, SMALL)],
                        ew_v.at[pl.ds(0, SMALL)])

        @pl.when(is_big)
        def _():
            pltpu.sync_copy(idx_hbm.at[pl.ds(ebase + SMALL * CE, 8 * CE)],
                            idx_v.at[pl.ds(SMALL * CE, 8 * CE)])
            pltpu.sync_copy(ew_hbm.at[pl.ds(start + SMALL, 8)],
                            ew_v.at[pl.ds(SMALL, 8)])

        if LEFT:
            @pl.when(is_last)
            def _():
                pltpu.sync_copy(
                    idx_hbm.at[pl.ds(ebase + SMALL * CE, LEFT * CE)],
                    idx_v.at[pl.ds(SMALL * CE, LEFT * CE)])
                pltpu.sync_copy(ew_hbm.at[pl.ds(start + SMALL, LEFT)],
                                ew_v.at[pl.ds(SMALL, LEFT)])

        lane_splat = [jnp.full((_LANES,), j, jnp.int32) for j in range(_LANES)]

        def issue(ci, slot):
            pltpu.async_copy(tbl_hbm.at[idx_v.at[pl.ds(ci * CE, CE)]],
                             rows_v.at[slot], sem.at[slot])

        def wait(slot):
            pltpu.make_async_copy(tbl_hbm.at[idx_v.at[pl.ds(0, CE)]],
                                  rows_v.at[slot], sem.at[slot]).wait()

        def compute(ci, slot):
            lrow = (ci - jnp.where(ci >= HALF, HALF, 0)) * CN
            for q in range(CN):
                acc = [jnp.zeros((_LANES,), jnp.float32) for _ in range(NCH)]
                for g in range(K // _LANES):
                    off = q * K + g * _LANES
                    ew = ew_v[ci, pl.ds(off, _LANES)]
                    for jj in range(_LANES):
                        e = q * K + g * _LANES + jj
                        wb = ew.at[lane_splat[jj]].get(
                            mode='promise_in_bounds')
                        for c in range(NCH):
                            r = rows_v[slot, e, pl.ds(c * _LANES, _LANES)]
                            acc[c] = acc[c] + wb * r
                row = lrow + q
                for c in range(NCH):
                    out_v[row, pl.ds(c * _LANES, _LANES)] = jnp.maximum(
                        acc[c], 0.0)

        issue(0, 0)
        issue(1, 1)
        issue(2, 2)
        issue(3, 3)

        def gbody(i, carry):
            @pl.when(i == HALF)
            def _():
                pltpu.sync_copy(out_v.at[pl.ds(0, HALF * CN)],
                                out_hbm.at[pl.ds(start * CN, HALF * CN)])

            slot = lax.rem(i, 5)
            wait(slot)
            compute(i, slot)
            nci = i + 4

            @pl.when(nci < n_mine)
            def _():
                issue(nci, lax.rem(nci, 5))
            return carry

        lax.fori_loop(0, n_mine, gbody, 0)

        tail_base = (start + HALF) * CN

        @pl.when(is_big)
        def _():
            pltpu.sync_copy(out_v.at[pl.ds(0, (BIG - HALF) * CN)],
                            out_hbm.at[pl.ds(tail_base, (BIG - HALF) * CN)])

        @pl.when(jnp.logical_and(jnp.logical_not(is_big),
                                 jnp.logical_not(is_last)))
        def _():
            pltpu.sync_copy(out_v.at[pl.ds(0, (SMALL - HALF) * CN)],
                            out_hbm.at[pl.ds(tail_base, (SMALL - HALF) * CN)])

        @pl.when(is_last)
        def _():
            pltpu.sync_copy(
                out_v.at[pl.ds(0, (NM_LAST - HALF) * CN)],
                out_hbm.at[pl.ds(tail_base, (NM_LAST - HALF) * CN)])

    return sc_gather


def kernel(node_feats, cond_feats, weights, params, coords_j, W_cond, b_cond,
           W_film, b_film):
    B, N, D = node_feats.shape
    K = weights.shape[2]
    OD = W_film.shape[0]
    E = B * N * K

    w_t = weights.transpose(0, 2, 3, 1).reshape(B, K, N)
    p_t = params.transpose(0, 2, 3, 1).reshape(B, K, N)
    tbl, ew = _film_call(node_feats, cond_feats, w_t, p_t, W_cond, b_cond,
                         W_film, b_film)
    sc = _make_sc_gather(B * N, OD, K, E)
    idx = (coords_j if coords_j.dtype == jnp.int32
           else coords_j.astype(jnp.int32))
    out = sc(tbl, idx, ew)
    return out.reshape(B, N, OD)
